# Initial kernel scaffold; baseline (speedup 1.0000x reference)
import jax
import jax.numpy as jnp
from jax.experimental import pallas as pl


def kernel(x, Wg, bg, W1, b1, W2, b2):
    raise NotImplementedError("write your pallas kernel here")


# speedup vs baseline: 3.2583x; 3.2583x over previous
import functools

import jax
import jax.numpy as jnp
from jax.experimental import pallas as pl

B, S, D, E, K = 2, 2048, 1024, 8, 2
DFF = D * 2
N = B * S
TT = 512
NT = N // TT


def _moe_tile(x_ref, wg_ref, bg_ref, w1_ref, b1_ref, w2_ref, b2_ref,
              out_ref, cnt_ref):
    x = x_ref[...]
    logits = jax.lax.dot_general(x, wg_ref[...],
                                 (((1,), (1,)), ((), ())),
                                 preferred_element_type=jnp.float32)
    logits = logits + bg_ref[...]
    m = jnp.max(logits, axis=1, keepdims=True)
    unn = jnp.exp(logits - m)
    p = unn / jnp.sum(unn, axis=1, keepdims=True)

    best = p[:, 0:1]
    besti = jnp.zeros((TT, 1), jnp.int32)
    for j in range(1, E):
        pj = p[:, j:j + 1]
        gt = pj > best
        besti = jnp.where(gt, j, besti)
        best = jnp.where(gt, pj, best)
    sec = jnp.full((TT, 1), -1.0, jnp.float32)
    seci = jnp.zeros((TT, 1), jnp.int32)
    for j in range(E):
        pj = jnp.where(besti == j, -1.0, p[:, j:j + 1])
        gt = pj > sec
        seci = jnp.where(gt, j, seci)
        sec = jnp.where(gt, pj, sec)

    w0 = jnp.where(besti == 0, p[:, 0:1], 0.0)
    w1 = jnp.where(seci == 1, p[:, 1:2], 0.0)

    xb = x.astype(jnp.bfloat16)
    acc = jnp.zeros((TT, D), jnp.float32)
    for i, w in ((0, w0), (1, w1)):
        h = jax.lax.dot_general(xb, w1_ref[i],
                                (((1,), (1,)), ((), ())),
                                preferred_element_type=jnp.float32)
        h = h + b1_ref[i]
        h = 0.5 * h * (1.0 + jax.lax.erf(h * 0.7071067811865476))
        y = jax.lax.dot_general(h.astype(jnp.bfloat16), w2_ref[i],
                                (((1,), (1,)), ((), ())),
                                preferred_element_type=jnp.float32)
        acc = acc + w * (y + b2_ref[i])
    out_ref[...] = acc

    cnt = (jnp.sum((besti < K).astype(jnp.int32))
           + jnp.sum((seci < K).astype(jnp.int32)))
    cnt_ref[...] = cnt.reshape(1, 1, 1)


@jax.jit
def _moe(x2d, Wg, bg, W1b, b1, W2b, b2):
    out, cnts = pl.pallas_call(
        _moe_tile,
        grid=(NT,),
        in_specs=[
            pl.BlockSpec((TT, D), lambda t: (t, 0)),
            pl.BlockSpec((E, D), lambda t: (0, 0)),
            pl.BlockSpec((1, E), lambda t: (0, 0)),
            pl.BlockSpec((K, DFF, D), lambda t: (0, 0, 0)),
            pl.BlockSpec((K, 1, DFF), lambda t: (0, 0, 0)),
            pl.BlockSpec((K, D, DFF), lambda t: (0, 0, 0)),
            pl.BlockSpec((K, 1, D), lambda t: (0, 0, 0)),
        ],
        out_specs=[
            pl.BlockSpec((TT, D), lambda t: (t, 0)),
            pl.BlockSpec((1, 1, 1), lambda t: (t, 0, 0)),
        ],
        out_shape=[
            jax.ShapeDtypeStruct((N, D), jnp.float32),
            jax.ShapeDtypeStruct((NT, 1, 1), jnp.int32),
        ],
    )(x2d, Wg, bg, W1b, b1, W2b, b2)
    return out, cnts


def kernel(x, Wg, bg, W1, b1, W2, b2):
    x2d = x.reshape(N, D)
    out, cnts = _moe(x2d, Wg, bg.reshape(1, E),
                     W1.astype(jnp.bfloat16), b1.reshape(K, 1, DFF),
                     W2.astype(jnp.bfloat16), b2.reshape(K, 1, D))
    total = cnts.sum()
    loss = jnp.where(total > 0, jnp.float32(0.0), jnp.float32(jnp.nan))
    return out.reshape(B, S, D), loss
